# Initial kernel scaffold; baseline (speedup 1.0000x reference)
import jax
import jax.numpy as jnp
from jax.experimental import pallas as pl


def kernel(x, edge_index, batch, W1, b1, g1, be1, W2, b2, g2, be2, W3, b3, g3, be3, fc1_W, fc1_b, fc2_W, fc2_b):
    raise NotImplementedError("write your pallas kernel here")


# speedup vs baseline: 13.8304x; 13.8304x over previous
import functools

import jax
import jax.numpy as jnp
from jax import lax
from jax.experimental import pallas as pl
from jax.experimental.pallas import tpu as pltpu
from jax.experimental.pallas import tpu_sc as plsc

EPS_ = 1e-5
NG_ = 32
LN_ = 128
GRP_ = 4
NSUB_ = 16
ZROWS_ = 128


def _round_up(a, m):
    return -(-a // m) * m


@functools.lru_cache(None)
def _make_sc_deg(n, rrows, na):
    mesh = plsc.VectorSubcoreMesh(core_axis_name="c", subcore_axis_name="s")
    del n
    rows_per_tile = rrows // (2 * NSUB_)
    groups = rows_per_tile // GRP_
    zslice = na // NSUB_
    zcnt = zslice // ZROWS_

    @functools.partial(
        pl.kernel,
        out_type=jax.ShapeDtypeStruct((2, na, 16), jnp.float32),
        mesh=mesh,
        compiler_params=pltpu.CompilerParams(use_tc_tiling_on_sc=False),
        scratch_types=[
            pltpu.VMEM((GRP_, LN_), jnp.int32),
            pltpu.VMEM((LN_, 16), jnp.float32),
            pltpu.VMEM((ZROWS_, 16), jnp.float32),
            pltpu.VMEM_SHARED((na, 16), jnp.float32),
        ],
    )
    def sc_deg(dstr, degp, dst_v, ones_v, zero_v, acc):
        core = lax.axis_index("c")
        sub = lax.axis_index("s")

        @pl.loop(0, LN_)
        def _(i):
            ones_v[i, pl.ds(0, 16)] = jnp.ones((16,), jnp.float32)

        @pl.loop(0, ZROWS_)
        def _(i):
            zero_v[i, pl.ds(0, 16)] = jnp.zeros((16,), jnp.float32)

        zbase = sub * zslice

        @pl.loop(0, zcnt)
        def _(k):
            pltpu.sync_copy(zero_v, acc.at[pl.ds(zbase + k * ZROWS_, ZROWS_)])

        plsc.subcore_barrier()

        wid = core * NSUB_ + sub
        base = wid * rows_per_tile

        @pl.loop(0, groups)
        def _(g):
            pltpu.sync_copy(dstr.at[pl.ds(base + g * GRP_, GRP_)], dst_v)
            for j in range(GRP_):
                pltpu.sync_copy(ones_v, acc.at[dst_v.at[j]], add=True)

        plsc.subcore_barrier()
        dbase = sub * zslice
        pltpu.sync_copy(acc.at[pl.ds(dbase, zslice)],
                        degp.at[core].at[pl.ds(dbase, zslice)])

    return sc_deg


@functools.lru_cache(None)
def _make_sc_agg(n, rrows, na):
    mesh = plsc.VectorSubcoreMesh(core_axis_name="c", subcore_axis_name="s")
    del n
    rows_per_tile = rrows // NSUB_
    groups = rows_per_tile // GRP_
    zslice = na // NSUB_
    zcnt = zslice // ZROWS_

    @functools.partial(
        pl.kernel,
        out_type=jax.ShapeDtypeStruct((2, na, 32), jnp.float32),
        mesh=mesh,
        compiler_params=pltpu.CompilerParams(use_tc_tiling_on_sc=False),
        scratch_types=[
            pltpu.VMEM((GRP_, LN_), jnp.int32),
            pltpu.VMEM((GRP_, LN_), jnp.int32),
            pltpu.VMEM((GRP_ * LN_, 32), jnp.float32),
            pltpu.VMEM((ZROWS_, 32), jnp.float32),
            pltpu.VMEM_SHARED((na, 32), jnp.float32),
        ],
    )
    def sc_agg(h2, srcr, dstr, agg2, src_v, dst_v, rows_v, zero_v, acc):
        core = lax.axis_index("c")
        sub = lax.axis_index("s")

        @pl.loop(0, ZROWS_)
        def _(i):
            zero_v[i, pl.ds(0, 16)] = jnp.zeros((16,), jnp.float32)
            zero_v[i, pl.ds(16, 16)] = jnp.zeros((16,), jnp.float32)

        zbase = sub * zslice

        @pl.loop(0, zcnt)
        def _(k):
            pltpu.sync_copy(zero_v, acc.at[pl.ds(zbase + k * ZROWS_, ZROWS_)])

        plsc.subcore_barrier()

        table = h2.at[core]
        base = sub * rows_per_tile

        @pl.loop(0, groups)
        def _(g):
            r0 = base + g * GRP_
            pltpu.sync_copy(srcr.at[pl.ds(r0, GRP_)], src_v)
            pltpu.sync_copy(dstr.at[pl.ds(r0, GRP_)], dst_v)
            for j in range(GRP_):
                pltpu.sync_copy(table.at[src_v.at[j]],
                                rows_v.at[pl.ds(j * LN_, LN_)])
            for j in range(GRP_):
                pltpu.sync_copy(rows_v.at[pl.ds(j * LN_, LN_)],
                                acc.at[dst_v.at[j]], add=True)

        plsc.subcore_barrier()
        dbase = sub * zslice
        pltpu.sync_copy(acc.at[pl.ds(dbase, zslice)],
                        agg2.at[core].at[pl.ds(dbase, zslice)])

    return sc_agg


def _tc_pre1(x, degp, w1, n, bn):
    grid = n // bn
    din = x.shape[1]

    def body(x_ref, d_ref, w_ref, h2_ref, dv_ref):
        deg = d_ref[0, :, 0:1] + d_ref[1, :, 0:1] + 1.0
        dv = lax.rsqrt(deg)
        h = jnp.dot(x_ref[...], w_ref[...],
                    preferred_element_type=jnp.float32) * dv
        h2_ref[0] = h[:, :32]
        h2_ref[1] = h[:, 32:]
        dv_ref[...] = dv

    return pl.pallas_call(
        body,
        grid=(grid,),
        in_specs=[
            pl.BlockSpec((bn, din), lambda i: (i, 0)),
            pl.BlockSpec((2, bn, 16), lambda i: (0, i, 0)),
            pl.BlockSpec(w1.shape, lambda i: (0, 0)),
        ],
        out_specs=[
            pl.BlockSpec((2, bn, 32), lambda i: (0, i, 0)),
            pl.BlockSpec((bn, 1), lambda i: (i, 0)),
        ],
        out_shape=[
            jax.ShapeDtypeStruct((2, n, 32), jnp.float32),
            jax.ShapeDtypeStruct((n, 1), jnp.float32),
        ],
    )(x, degp, w1)


def _tc_post(agg2, h2, dinv, b, n, bn):
    grid = n // bn

    def body(a_ref, h_ref, dv_ref, b_ref, t_ref, st_ref, acc_ref):
        i = pl.program_id(0)
        a = jnp.concatenate([a_ref[0] + h_ref[0], a_ref[1] + h_ref[1]],
                            axis=1)
        t = dv_ref[...] * a + b_ref[...]
        t_ref[...] = t
        part = jnp.stack([jnp.sum(t, axis=0), jnp.sum(t * t, axis=0)])

        @pl.when(i == 0)
        def _():
            acc_ref[...] = part

        @pl.when(i > 0)
        def _():
            acc_ref[...] += part

        @pl.when(i == grid - 1)
        def _():
            st_ref[...] = acc_ref[...]

    return pl.pallas_call(
        body,
        grid=(grid,),
        in_specs=[
            pl.BlockSpec((2, bn, 32), lambda i: (0, i, 0)),
            pl.BlockSpec((2, bn, 32), lambda i: (0, i, 0)),
            pl.BlockSpec((bn, 1), lambda i: (i, 0)),
            pl.BlockSpec((1, 64), lambda i: (0, 0)),
        ],
        out_specs=[
            pl.BlockSpec((bn, 64), lambda i: (i, 0)),
            pl.BlockSpec((2, 64), lambda i: (0, 0)),
        ],
        out_shape=[
            jax.ShapeDtypeStruct((n, 64), jnp.float32),
            jax.ShapeDtypeStruct((2, 64), jnp.float32),
        ],
        scratch_shapes=[pltpu.VMEM((2, 64), jnp.float32)],
    )(agg2, h2, dinv, b)


def _tc_pre23(t, st, g, be, w, dinv, n, bn):
    grid = n // bn
    inv_n = 1.0 / n

    def body(t_ref, st_ref, g_ref, be_ref, w_ref, dv_ref, h2_ref):
        m = st_ref[0:1, :] * inv_n
        v = st_ref[1:2, :] * inv_n - m * m
        rstd = lax.rsqrt(v + EPS_)
        act = jnp.maximum((t_ref[...] - m) * rstd * g_ref[...] + be_ref[...],
                          0.0)
        h = jnp.dot(act, w_ref[...],
                    preferred_element_type=jnp.float32) * dv_ref[...]
        h2_ref[0] = h[:, :32]
        h2_ref[1] = h[:, 32:]

    return pl.pallas_call(
        body,
        grid=(grid,),
        in_specs=[
            pl.BlockSpec((bn, 64), lambda i: (i, 0)),
            pl.BlockSpec((2, 64), lambda i: (0, 0)),
            pl.BlockSpec((1, 64), lambda i: (0, 0)),
            pl.BlockSpec((1, 64), lambda i: (0, 0)),
            pl.BlockSpec((64, 64), lambda i: (0, 0)),
            pl.BlockSpec((bn, 1), lambda i: (i, 0)),
        ],
        out_specs=[pl.BlockSpec((2, bn, 32), lambda i: (0, i, 0))],
        out_shape=[jax.ShapeDtypeStruct((2, n, 32), jnp.float32)],
    )(t, st, g, be, w, dinv)[0]


def _tc_final(t, st, g, be, batch2, f1w, f1b, f2w, f2b, n, bn):
    grid = n // bn
    inv_n = 1.0 / n

    def body(t_ref, st_ref, g_ref, be_ref, bt_ref, f1w_ref, f1b_ref,
             f2w_ref, f2b_ref, o_ref, ps_ref, pc_ref):
        i = pl.program_id(0)

        @pl.when(i == 0)
        def _():
            ps_ref[...] = jnp.zeros_like(ps_ref)
            pc_ref[...] = jnp.zeros_like(pc_ref)

        m = st_ref[0:1, :] * inv_n
        v = st_ref[1:2, :] * inv_n - m * m
        rstd = lax.rsqrt(v + EPS_)
        act = jnp.maximum((t_ref[...] - m) * rstd * g_ref[...] + be_ref[...],
                          0.0)
        gid = lax.broadcasted_iota(jnp.int32, (bn, NG_), 1)
        mask = (bt_ref[...] == gid).astype(jnp.float32)
        ps_ref[...] += lax.dot_general(mask, act, (((0,), (0,)), ((), ())),
                                       preferred_element_type=jnp.float32)
        pc_ref[...] += lax.dot_general(mask, jnp.ones((bn, 1), jnp.float32),
                                       (((0,), (0,)), ((), ())),
                                       preferred_element_type=jnp.float32)

        @pl.when(i == grid - 1)
        def _():
            pooled = ps_ref[...] / jnp.maximum(pc_ref[...], 1.0)
            z = jnp.maximum(jnp.dot(pooled, f1w_ref[...],
                                    preferred_element_type=jnp.float32)
                            + f1b_ref[...], 0.0)
            o_ref[...] = jnp.dot(z, f2w_ref[...],
                                 preferred_element_type=jnp.float32) \
                + f2b_ref[...]

    return pl.pallas_call(
        body,
        grid=(grid,),
        in_specs=[
            pl.BlockSpec((bn, 64), lambda i: (i, 0)),
            pl.BlockSpec((2, 64), lambda i: (0, 0)),
            pl.BlockSpec((1, 64), lambda i: (0, 0)),
            pl.BlockSpec((1, 64), lambda i: (0, 0)),
            pl.BlockSpec((bn, 1), lambda i: (i, 0)),
            pl.BlockSpec((64, 64), lambda i: (0, 0)),
            pl.BlockSpec((1, 64), lambda i: (0, 0)),
            pl.BlockSpec((64, 1), lambda i: (0, 0)),
            pl.BlockSpec((1, 1), lambda i: (0, 0)),
        ],
        out_specs=[pl.BlockSpec((NG_, 1), lambda i: (0, 0))],
        out_shape=[jax.ShapeDtypeStruct((NG_, 1), jnp.float32)],
        scratch_shapes=[
            pltpu.VMEM((NG_, 64), jnp.float32),
            pltpu.VMEM((NG_, 1), jnp.float32),
        ],
    )(t, st, g, be, batch2, f1w, f1b, f2w, f2b)[0]


def kernel(x, edge_index, batch, W1, b1, g1, be1, W2, b2, g2, be2,
           W3, b3, g3, be3, fc1_W, fc1_b, fc2_W, fc2_b):
    n = x.shape[0]
    e = edge_index.shape[1]
    bn = 2000
    assert n % bn == 0 and n % NSUB_ == 0

    rrows = _round_up(-(-e // LN_), 2 * NSUB_ * GRP_)
    epad = rrows * LN_ - e
    src = edge_index[0]
    dst = edge_index[1]
    if epad:
        src = jnp.concatenate([src, jnp.zeros((epad,), jnp.int32)])
        dst = jnp.concatenate([dst, jnp.full((epad,), n, jnp.int32)])
    srcr = src.reshape(rrows, LN_)
    dstr = dst.reshape(rrows, LN_)

    na = _round_up(n + 8, NSUB_ * ZROWS_)

    degp = _make_sc_deg(n, rrows, na)(dstr)
    h2, dinv = _tc_pre1(x, degp, W1, n, bn)

    sc_agg = _make_sc_agg(n, rrows, na)
    b1r, b2r, b3r = (b.reshape(1, 64) for b in (b1, b2, b3))
    g1r, g2r, g3r = (g.reshape(1, 64) for g in (g1, g2, g3))
    be1r, be2r, be3r = (b.reshape(1, 64) for b in (be1, be2, be3))

    agg2 = sc_agg(h2, srcr, dstr)
    t1, st1 = _tc_post(agg2, h2, dinv, b1r, n, bn)

    h2 = _tc_pre23(t1, st1, g1r, be1r, W2, dinv, n, bn)
    agg2 = sc_agg(h2, srcr, dstr)
    t2, st2 = _tc_post(agg2, h2, dinv, b2r, n, bn)

    h2 = _tc_pre23(t2, st2, g2r, be2r, W3, dinv, n, bn)
    agg2 = sc_agg(h2, srcr, dstr)
    t3, st3 = _tc_post(agg2, h2, dinv, b3r, n, bn)

    return _tc_final(t3, st3, g3r, be3r, batch.reshape(n, 1),
                     fc1_W, fc1_b.reshape(1, 64), fc2_W,
                     fc2_b.reshape(1, 1), n, bn)


# speedup vs baseline: 15.7665x; 1.1400x over previous
import functools

import jax
import jax.numpy as jnp
from jax import lax
from jax.experimental import pallas as pl
from jax.experimental.pallas import tpu as pltpu
from jax.experimental.pallas import tpu_sc as plsc

EPS_ = 1e-5
NG_ = 32
LN_ = 128
GRP_ = 2
BLKG_ = 8
DGRP_ = 4
NSUB_ = 16
ZROWS_ = 128


def _round_up(a, m):
    return -(-a // m) * m


def _zero_offsets(zslice):
    offs = [k * ZROWS_ for k in range(zslice // ZROWS_)]
    if zslice % ZROWS_:
        offs.append(zslice - ZROWS_)
    return offs


@functools.lru_cache(None)
def _make_sc_deg(n, rrows, na):
    mesh = plsc.VectorSubcoreMesh(core_axis_name="c", subcore_axis_name="s")
    del n
    rows_per_tile = rrows // (2 * NSUB_)
    groups = rows_per_tile // DGRP_
    zslice = na // NSUB_

    @functools.partial(
        pl.kernel,
        out_type=jax.ShapeDtypeStruct((2, na, 16), jnp.float32),
        mesh=mesh,
        compiler_params=pltpu.CompilerParams(use_tc_tiling_on_sc=False),
        scratch_types=[
            pltpu.VMEM((DGRP_, 2, LN_), jnp.int32),
            pltpu.VMEM((LN_, 16), jnp.float32),
            pltpu.VMEM((ZROWS_, 16), jnp.float32),
            pltpu.VMEM_SHARED((na, 16), jnp.float32),
            pltpu.SemaphoreType.DMA,
        ],
    )
    def sc_deg(srcdst, degp, dst_v, ones_v, zero_v, acc, sem):
        core = lax.axis_index("c")
        sub = lax.axis_index("s")

        @pl.loop(0, LN_)
        def _(i):
            ones_v[i, pl.ds(0, 16)] = jnp.ones((16,), jnp.float32)

        @pl.loop(0, ZROWS_)
        def _(i):
            zero_v[i, pl.ds(0, 16)] = jnp.zeros((16,), jnp.float32)

        zbase = sub * zslice
        offs = _zero_offsets(zslice)
        for o in offs:
            pltpu.async_copy(zero_v, acc.at[pl.ds(zbase + o, ZROWS_)], sem)
        for o in offs:
            pltpu.make_async_copy(zero_v, acc.at[pl.ds(zbase + o, ZROWS_)],
                                  sem).wait()

        plsc.subcore_barrier()

        wid = core * NSUB_ + sub
        base = wid * rows_per_tile

        @pl.loop(0, groups)
        def _(g):
            pltpu.sync_copy(srcdst.at[pl.ds(base + g * DGRP_, DGRP_)], dst_v)
            for j in range(DGRP_):
                pltpu.async_copy(ones_v, acc.at[dst_v.at[j, 1]], sem,
                                 add=True)
            for j in range(DGRP_):
                pltpu.make_async_copy(ones_v, acc.at[dst_v.at[j, 1]],
                                      sem).wait()

        plsc.subcore_barrier()
        dbase = sub * zslice
        pltpu.sync_copy(acc.at[pl.ds(dbase, zslice)],
                        degp.at[core].at[pl.ds(dbase, zslice)])

    return sc_deg


@functools.lru_cache(None)
def _make_sc_agg(n, rrows, na):
    mesh = plsc.VectorSubcoreMesh(core_axis_name="c", subcore_axis_name="s")
    del n
    rows_per_tile = rrows // NSUB_
    groups = rows_per_tile // GRP_
    nblk = groups // BLKG_
    assert nblk >= 3 and nblk % 2 == 1 and groups % BLKG_ == 0
    brows = BLKG_ * GRP_
    zslice = na // NSUB_

    @functools.partial(
        pl.kernel,
        out_type=jax.ShapeDtypeStruct((2, na, 32), jnp.float32),
        mesh=mesh,
        compiler_params=pltpu.CompilerParams(use_tc_tiling_on_sc=False),
        scratch_types=[
            pltpu.VMEM((brows, 2, LN_), jnp.int32),
            pltpu.VMEM((brows, 2, LN_), jnp.int32),
            pltpu.VMEM((GRP_ * LN_, 32), jnp.float32),
            pltpu.VMEM((GRP_ * LN_, 32), jnp.float32),
            pltpu.VMEM((ZROWS_, 32), jnp.float32),
            pltpu.VMEM_SHARED((na, 32), jnp.float32),
            pltpu.SemaphoreType.DMA,
            pltpu.SemaphoreType.DMA,
            pltpu.SemaphoreType.DMA,
            pltpu.SemaphoreType.DMA,
            pltpu.SemaphoreType.DMA,
            pltpu.SemaphoreType.DMA,
        ],
    )
    def sc_agg(h2, srcdst, agg2, idx0, idx1, rows0, rows1, zero_v, acc,
               si0, si1, sg0, sg1, ss0, ss1):
        core = lax.axis_index("c")
        sub = lax.axis_index("s")
        idx = (idx0, idx1)
        rows = (rows0, rows1)
        si = (si0, si1)
        sg = (sg0, sg1)
        ss = (ss0, ss1)
        table = h2.at[core]
        base = sub * rows_per_tile

        @pl.loop(0, ZROWS_)
        def _(i):
            zero_v[i, pl.ds(0, 16)] = jnp.zeros((16,), jnp.float32)
            zero_v[i, pl.ds(16, 16)] = jnp.zeros((16,), jnp.float32)

        zbase = sub * zslice
        offs = _zero_offsets(zslice)
        for o in offs:
            pltpu.async_copy(zero_v, acc.at[pl.ds(zbase + o, ZROWS_)], sg0)
        for o in offs:
            pltpu.make_async_copy(zero_v, acc.at[pl.ds(zbase + o, ZROWS_)],
                                  sg0).wait()

        plsc.subcore_barrier()

        def idx_start(q, ib):
            pltpu.async_copy(srcdst.at[pl.ds(base + q * brows, brows)],
                             idx[ib], si[ib])

        def idx_wait(ib):
            pltpu.make_async_copy(srcdst.at[pl.ds(base, brows)], idx[ib],
                                  si[ib]).wait()

        def gath_start(ib, k, rb):
            for j in range(GRP_):
                pltpu.async_copy(table.at[idx[ib].at[k * GRP_ + j, 0]],
                                 rows[rb].at[pl.ds(j * LN_, LN_)], sg[rb])

        def gath_wait(rb):
            pltpu.make_async_copy(table.at[pl.ds(0, GRP_ * LN_)], rows[rb],
                                  sg[rb]).wait()

        def scat_start(ib, k, rb):
            for j in range(GRP_):
                pltpu.async_copy(rows[rb].at[pl.ds(j * LN_, LN_)],
                                 acc.at[idx[ib].at[k * GRP_ + j, 1]],
                                 ss[rb], add=True)

        def scat_wait(rb):
            pltpu.make_async_copy(table.at[pl.ds(0, GRP_ * LN_)], rows[rb],
                                  ss[rb]).wait()

        def block(q, ib, first=False, prefetch=True, last=False):
            for k in range(BLKG_):
                rb = k % 2
                gath_wait(rb)
                scat_start(ib, k, rb)
                if last and k == BLKG_ - 1:
                    continue
                if k == BLKG_ - 1:
                    idx_wait(ib ^ 1)
                if not (first and k == 0):
                    scat_wait(rb ^ 1)
                if k == 1 and prefetch:
                    idx_start(q + 1, ib ^ 1)
                if k == BLKG_ - 1:
                    gath_start(ib ^ 1, 0, rb ^ 1)
                else:
                    gath_start(ib, k + 1, rb ^ 1)

        idx_start(0, 0)
        idx_start(1, 1)
        idx_wait(0)
        gath_start(0, 0, 0)

        block(0, 0, first=True, prefetch=False)

        @pl.loop(1, nblk - 2, step=2)
        def _(p):
            block(p, 1)
            block(p + 1, 0)

        block(nblk - 2, 1)
        block(nblk - 1, 0, prefetch=False, last=True)

        scat_wait(0)
        scat_wait(1)

        plsc.subcore_barrier()
        dbase = sub * zslice
        pltpu.sync_copy(acc.at[pl.ds(dbase, zslice)],
                        agg2.at[core].at[pl.ds(dbase, zslice)])

    return sc_agg


def _tc_pre1(x, degp, w1, n, bn):
    grid = n // bn
    din = x.shape[1]

    def body(x_ref, d_ref, w_ref, h2_ref, dv_ref):
        deg = d_ref[0, :, 0:1] + d_ref[1, :, 0:1] + 1.0
        dv = lax.rsqrt(deg)
        h = jnp.dot(x_ref[...], w_ref[...],
                    preferred_element_type=jnp.float32) * dv
        h2_ref[0] = h[:, :32]
        h2_ref[1] = h[:, 32:]
        dv_ref[...] = dv

    return pl.pallas_call(
        body,
        grid=(grid,),
        in_specs=[
            pl.BlockSpec((bn, din), lambda i: (i, 0)),
            pl.BlockSpec((2, bn, 16), lambda i: (0, i, 0)),
            pl.BlockSpec(w1.shape, lambda i: (0, 0)),
        ],
        out_specs=[
            pl.BlockSpec((2, bn, 32), lambda i: (0, i, 0)),
            pl.BlockSpec((bn, 1), lambda i: (i, 0)),
        ],
        out_shape=[
            jax.ShapeDtypeStruct((2, n, 32), jnp.float32),
            jax.ShapeDtypeStruct((n, 1), jnp.float32),
        ],
    )(x, degp, w1)


def _tc_post(agg2, h2, dinv, b, n, bn):
    grid = n // bn

    def body(a_ref, h_ref, dv_ref, b_ref, t_ref, st_ref, acc_ref):
        i = pl.program_id(0)
        a = jnp.concatenate([a_ref[0] + h_ref[0], a_ref[1] + h_ref[1]],
                            axis=1)
        t = dv_ref[...] * a + b_ref[...]
        t_ref[...] = t
        part = jnp.stack([jnp.sum(t, axis=0), jnp.sum(t * t, axis=0)])

        @pl.when(i == 0)
        def _():
            acc_ref[...] = part

        @pl.when(i > 0)
        def _():
            acc_ref[...] += part

        @pl.when(i == grid - 1)
        def _():
            st_ref[...] = acc_ref[...]

    return pl.pallas_call(
        body,
        grid=(grid,),
        in_specs=[
            pl.BlockSpec((2, bn, 32), lambda i: (0, i, 0)),
            pl.BlockSpec((2, bn, 32), lambda i: (0, i, 0)),
            pl.BlockSpec((bn, 1), lambda i: (i, 0)),
            pl.BlockSpec((1, 64), lambda i: (0, 0)),
        ],
        out_specs=[
            pl.BlockSpec((bn, 64), lambda i: (i, 0)),
            pl.BlockSpec((2, 64), lambda i: (0, 0)),
        ],
        out_shape=[
            jax.ShapeDtypeStruct((n, 64), jnp.float32),
            jax.ShapeDtypeStruct((2, 64), jnp.float32),
        ],
        scratch_shapes=[pltpu.VMEM((2, 64), jnp.float32)],
    )(agg2, h2, dinv, b)


def _tc_pre23(t, st, g, be, w, dinv, n, bn):
    grid = n // bn
    inv_n = 1.0 / n

    def body(t_ref, st_ref, g_ref, be_ref, w_ref, dv_ref, h2_ref):
        m = st_ref[0:1, :] * inv_n
        v = st_ref[1:2, :] * inv_n - m * m
        rstd = lax.rsqrt(v + EPS_)
        act = jnp.maximum((t_ref[...] - m) * rstd * g_ref[...] + be_ref[...],
                          0.0)
        h = jnp.dot(act, w_ref[...],
                    preferred_element_type=jnp.float32) * dv_ref[...]
        h2_ref[0] = h[:, :32]
        h2_ref[1] = h[:, 32:]

    return pl.pallas_call(
        body,
        grid=(grid,),
        in_specs=[
            pl.BlockSpec((bn, 64), lambda i: (i, 0)),
            pl.BlockSpec((2, 64), lambda i: (0, 0)),
            pl.BlockSpec((1, 64), lambda i: (0, 0)),
            pl.BlockSpec((1, 64), lambda i: (0, 0)),
            pl.BlockSpec((64, 64), lambda i: (0, 0)),
            pl.BlockSpec((bn, 1), lambda i: (i, 0)),
        ],
        out_specs=[pl.BlockSpec((2, bn, 32), lambda i: (0, i, 0))],
        out_shape=[jax.ShapeDtypeStruct((2, n, 32), jnp.float32)],
    )(t, st, g, be, w, dinv)[0]


def _tc_final(t, st, g, be, batch2, f1w, f1b, f2w, f2b, n, bn):
    grid = n // bn
    inv_n = 1.0 / n

    def body(t_ref, st_ref, g_ref, be_ref, bt_ref, f1w_ref, f1b_ref,
             f2w_ref, f2b_ref, o_ref, ps_ref, pc_ref):
        i = pl.program_id(0)

        @pl.when(i == 0)
        def _():
            ps_ref[...] = jnp.zeros_like(ps_ref)
            pc_ref[...] = jnp.zeros_like(pc_ref)

        m = st_ref[0:1, :] * inv_n
        v = st_ref[1:2, :] * inv_n - m * m
        rstd = lax.rsqrt(v + EPS_)
        act = jnp.maximum((t_ref[...] - m) * rstd * g_ref[...] + be_ref[...],
                          0.0)
        gid = lax.broadcasted_iota(jnp.int32, (bn, NG_), 1)
        mask = (bt_ref[...] == gid).astype(jnp.float32)
        ps_ref[...] += lax.dot_general(mask, act, (((0,), (0,)), ((), ())),
                                       preferred_element_type=jnp.float32)
        pc_ref[...] += lax.dot_general(mask, jnp.ones((bn, 1), jnp.float32),
                                       (((0,), (0,)), ((), ())),
                                       preferred_element_type=jnp.float32)

        @pl.when(i == grid - 1)
        def _():
            pooled = ps_ref[...] / jnp.maximum(pc_ref[...], 1.0)
            z = jnp.maximum(jnp.dot(pooled, f1w_ref[...],
                                    preferred_element_type=jnp.float32)
                            + f1b_ref[...], 0.0)
            o_ref[...] = jnp.dot(z, f2w_ref[...],
                                 preferred_element_type=jnp.float32) \
                + f2b_ref[...]

    return pl.pallas_call(
        body,
        grid=(grid,),
        in_specs=[
            pl.BlockSpec((bn, 64), lambda i: (i, 0)),
            pl.BlockSpec((2, 64), lambda i: (0, 0)),
            pl.BlockSpec((1, 64), lambda i: (0, 0)),
            pl.BlockSpec((1, 64), lambda i: (0, 0)),
            pl.BlockSpec((bn, 1), lambda i: (i, 0)),
            pl.BlockSpec((64, 64), lambda i: (0, 0)),
            pl.BlockSpec((1, 64), lambda i: (0, 0)),
            pl.BlockSpec((64, 1), lambda i: (0, 0)),
            pl.BlockSpec((1, 1), lambda i: (0, 0)),
        ],
        out_specs=[pl.BlockSpec((NG_, 1), lambda i: (0, 0))],
        out_shape=[jax.ShapeDtypeStruct((NG_, 1), jnp.float32)],
        scratch_shapes=[
            pltpu.VMEM((NG_, 64), jnp.float32),
            pltpu.VMEM((NG_, 1), jnp.float32),
        ],
    )(t, st, g, be, batch2, f1w, f1b, f2w, f2b)[0]


def kernel(x, edge_index, batch, W1, b1, g1, be1, W2, b2, g2, be2,
           W3, b3, g3, be3, fc1_W, fc1_b, fc2_W, fc2_b):
    n = x.shape[0]
    e = edge_index.shape[1]
    bn = 2000
    assert n % bn == 0 and n % NSUB_ == 0

    rrows = _round_up(-(-e // LN_), NSUB_ * GRP_ * BLKG_)
    if (rrows // (NSUB_ * GRP_ * BLKG_)) % 2 == 0:
        rrows += NSUB_ * GRP_ * BLKG_
    epad = rrows * LN_ - e
    src = edge_index[0]
    dst = edge_index[1]
    if epad:
        src = jnp.concatenate([src, jnp.zeros((epad,), jnp.int32)])
        dst = jnp.concatenate([dst, jnp.full((epad,), n, jnp.int32)])
    srcdst = jnp.stack([src.reshape(rrows, LN_), dst.reshape(rrows, LN_)],
                       axis=1)

    na = _round_up(n + 8, LN_)

    degp = _make_sc_deg(n, rrows, na)(srcdst)
    h2, dinv = _tc_pre1(x, degp, W1, n, bn)

    sc_agg = _make_sc_agg(n, rrows, na)
    b1r, b2r, b3r = (b.reshape(1, 64) for b in (b1, b2, b3))
    g1r, g2r, g3r = (g.reshape(1, 64) for g in (g1, g2, g3))
    be1r, be2r, be3r = (b.reshape(1, 64) for b in (be1, be2, be3))

    agg2 = sc_agg(h2, srcdst)
    t1, st1 = _tc_post(agg2, h2, dinv, b1r, n, bn)

    h2 = _tc_pre23(t1, st1, g1r, be1r, W2, dinv, n, bn)
    agg2 = sc_agg(h2, srcdst)
    t2, st2 = _tc_post(agg2, h2, dinv, b2r, n, bn)

    h2 = _tc_pre23(t2, st2, g2r, be2r, W3, dinv, n, bn)
    agg2 = sc_agg(h2, srcdst)
    t3, st3 = _tc_post(agg2, h2, dinv, b3r, n, bn)

    return _tc_final(t3, st3, g3r, be3r, batch.reshape(n, 1),
                     fc1_W, fc1_b.reshape(1, 64), fc2_W,
                     fc2_b.reshape(1, 1), n, bn)


# speedup vs baseline: 16.7611x; 1.0631x over previous
import functools

import jax
import jax.numpy as jnp
from jax import lax
from jax.experimental import pallas as pl
from jax.experimental.pallas import tpu as pltpu
from jax.experimental.pallas import tpu_sc as plsc

EPS_ = 1e-5
NG_ = 32
LN_ = 256
BLKG_ = 8
DGRP_ = 4
NSUB_ = 16
ZROWS_ = 128


def _round_up(a, m):
    return -(-a // m) * m


def _zero_offsets(zslice):
    offs = [k * ZROWS_ for k in range(zslice // ZROWS_)]
    if zslice % ZROWS_:
        offs.append(zslice - ZROWS_)
    return offs


@functools.lru_cache(None)
def _make_sc_deg(n, rrows, na):
    mesh = plsc.VectorSubcoreMesh(core_axis_name="c", subcore_axis_name="s")
    del n
    rows_per_tile = rrows // (2 * NSUB_)
    groups = rows_per_tile // DGRP_
    zslice = na // NSUB_

    @functools.partial(
        pl.kernel,
        out_type=jax.ShapeDtypeStruct((2, na, 16), jnp.float32),
        mesh=mesh,
        compiler_params=pltpu.CompilerParams(use_tc_tiling_on_sc=False),
        scratch_types=[
            pltpu.VMEM((DGRP_, 2, LN_), jnp.int32),
            pltpu.VMEM((LN_, 16), jnp.float32),
            pltpu.VMEM((ZROWS_, 16), jnp.float32),
            pltpu.VMEM_SHARED((na, 16), jnp.float32),
            pltpu.SemaphoreType.DMA,
        ],
    )
    def sc_deg(srcdst, degp, dst_v, ones_v, zero_v, acc, sem):
        core = lax.axis_index("c")
        sub = lax.axis_index("s")

        @pl.loop(0, LN_)
        def _(i):
            ones_v[i, pl.ds(0, 16)] = jnp.ones((16,), jnp.float32)

        @pl.loop(0, ZROWS_)
        def _(i):
            zero_v[i, pl.ds(0, 16)] = jnp.zeros((16,), jnp.float32)

        zbase = sub * zslice
        offs = _zero_offsets(zslice)
        for o in offs:
            pltpu.async_copy(zero_v, acc.at[pl.ds(zbase + o, ZROWS_)], sem)
        for o in offs:
            pltpu.make_async_copy(zero_v, acc.at[pl.ds(zbase + o, ZROWS_)],
                                  sem).wait()

        plsc.subcore_barrier()

        wid = core * NSUB_ + sub
        base = wid * rows_per_tile

        @pl.loop(0, groups)
        def _(g):
            pltpu.sync_copy(srcdst.at[pl.ds(base + g * DGRP_, DGRP_)], dst_v)
            for j in range(DGRP_):
                pltpu.async_copy(ones_v, acc.at[dst_v.at[j, 1]], sem,
                                 add=True)
            for j in range(DGRP_):
                pltpu.make_async_copy(ones_v, acc.at[dst_v.at[j, 1]],
                                      sem).wait()

        plsc.subcore_barrier()
        dbase = sub * zslice
        pltpu.sync_copy(acc.at[pl.ds(dbase, zslice)],
                        degp.at[core].at[pl.ds(dbase, zslice)])

    return sc_deg


@functools.lru_cache(None)
def _make_sc_agg(n, rrows, na):
    mesh = plsc.VectorSubcoreMesh(core_axis_name="c", subcore_axis_name="s")
    del n
    rows_per_tile = rrows // NSUB_
    groups = rows_per_tile
    nblk = groups // BLKG_
    assert nblk >= 3 and nblk % 2 == 1 and groups % BLKG_ == 0
    brows = BLKG_
    zslice = na // NSUB_

    @functools.partial(
        pl.kernel,
        out_type=jax.ShapeDtypeStruct((2, na, 32), jnp.float32),
        mesh=mesh,
        compiler_params=pltpu.CompilerParams(use_tc_tiling_on_sc=False),
        scratch_types=[
            pltpu.VMEM((brows, 2, LN_), jnp.int32),
            pltpu.VMEM((brows, 2, LN_), jnp.int32),
            pltpu.VMEM((LN_, 32), jnp.float32),
            pltpu.VMEM((LN_, 32), jnp.float32),
            pltpu.VMEM((ZROWS_, 32), jnp.float32),
            pltpu.VMEM_SHARED((na, 32), jnp.float32),
            pltpu.SemaphoreType.DMA,
            pltpu.SemaphoreType.DMA,
            pltpu.SemaphoreType.DMA,
            pltpu.SemaphoreType.DMA,
            pltpu.SemaphoreType.DMA,
            pltpu.SemaphoreType.DMA,
        ],
    )
    def sc_agg(h2, srcdst, agg2, idx0, idx1, rows0, rows1, zero_v, acc,
               si0, si1, sg0, sg1, ss0, ss1):
        core = lax.axis_index("c")
        sub = lax.axis_index("s")
        idx = (idx0, idx1)
        rows = (rows0, rows1)
        si = (si0, si1)
        sg = (sg0, sg1)
        ss = (ss0, ss1)
        table = h2.at[core]
        base = sub * rows_per_tile

        @pl.loop(0, ZROWS_)
        def _(i):
            zero_v[i, pl.ds(0, 16)] = jnp.zeros((16,), jnp.float32)
            zero_v[i, pl.ds(16, 16)] = jnp.zeros((16,), jnp.float32)

        zbase = sub * zslice
        offs = _zero_offsets(zslice)
        for o in offs:
            pltpu.async_copy(zero_v, acc.at[pl.ds(zbase + o, ZROWS_)], sg0)
        for o in offs:
            pltpu.make_async_copy(zero_v, acc.at[pl.ds(zbase + o, ZROWS_)],
                                  sg0).wait()

        plsc.subcore_barrier()

        def idx_start(q, ib):
            pltpu.async_copy(srcdst.at[pl.ds(base + q * brows, brows)],
                             idx[ib], si[ib])

        def idx_wait(ib):
            pltpu.make_async_copy(srcdst.at[pl.ds(base, brows)], idx[ib],
                                  si[ib]).wait()

        def gath_start(ib, k, rb):
            pltpu.async_copy(table.at[idx[ib].at[k, 0]], rows[rb], sg[rb])

        def gath_wait(rb):
            pltpu.make_async_copy(table.at[pl.ds(0, LN_)], rows[rb],
                                  sg[rb]).wait()

        def scat_start(ib, k, rb):
            pltpu.async_copy(rows[rb], acc.at[idx[ib].at[k, 1]], ss[rb],
                             add=True)

        def scat_wait(rb):
            pltpu.make_async_copy(table.at[pl.ds(0, LN_)], rows[rb],
                                  ss[rb]).wait()

        def block(q, ib, first=False, prefetch=True, last=False):
            for k in range(BLKG_):
                rb = k % 2
                gath_wait(rb)
                scat_start(ib, k, rb)
                if last and k == BLKG_ - 1:
                    continue
                if k == BLKG_ - 1:
                    idx_wait(ib ^ 1)
                if not (first and k == 0):
                    scat_wait(rb ^ 1)
                if k == 1 and prefetch:
                    idx_start(q + 1, ib ^ 1)
                if k == BLKG_ - 1:
                    gath_start(ib ^ 1, 0, rb ^ 1)
                else:
                    gath_start(ib, k + 1, rb ^ 1)

        idx_start(0, 0)
        idx_start(1, 1)
        idx_wait(0)
        gath_start(0, 0, 0)

        block(0, 0, first=True, prefetch=False)

        @pl.loop(1, nblk - 2, step=2)
        def _(p):
            block(p, 1)
            block(p + 1, 0)

        block(nblk - 2, 1)
        block(nblk - 1, 0, prefetch=False, last=True)

        scat_wait(0)
        scat_wait(1)

        plsc.subcore_barrier()
        dbase = sub * zslice
        pltpu.sync_copy(acc.at[pl.ds(dbase, zslice)],
                        agg2.at[core].at[pl.ds(dbase, zslice)])

    return sc_agg


def _tc_pre1(x, degp, w1, n, bn):
    grid = n // bn
    din = x.shape[1]

    def body(x_ref, d_ref, w_ref, h2_ref, dv_ref):
        deg = d_ref[0, :, 0:1] + d_ref[1, :, 0:1] + 1.0
        dv = lax.rsqrt(deg)
        h = jnp.dot(x_ref[...], w_ref[...],
                    preferred_element_type=jnp.float32) * dv
        h2_ref[0] = h[:, :32]
        h2_ref[1] = h[:, 32:]
        dv_ref[...] = dv

    return pl.pallas_call(
        body,
        grid=(grid,),
        in_specs=[
            pl.BlockSpec((bn, din), lambda i: (i, 0)),
            pl.BlockSpec((2, bn, 16), lambda i: (0, i, 0)),
            pl.BlockSpec(w1.shape, lambda i: (0, 0)),
        ],
        out_specs=[
            pl.BlockSpec((2, bn, 32), lambda i: (0, i, 0)),
            pl.BlockSpec((bn, 1), lambda i: (i, 0)),
        ],
        out_shape=[
            jax.ShapeDtypeStruct((2, n, 32), jnp.float32),
            jax.ShapeDtypeStruct((n, 1), jnp.float32),
        ],
    )(x, degp, w1)


def _tc_post(agg2, h2, dinv, b, n, bn):
    grid = n // bn

    def body(a_ref, h_ref, dv_ref, b_ref, t_ref, st_ref, acc_ref):
        i = pl.program_id(0)
        a = jnp.concatenate([a_ref[0] + h_ref[0], a_ref[1] + h_ref[1]],
                            axis=1)
        t = dv_ref[...] * a + b_ref[...]
        t_ref[...] = t
        part = jnp.stack([jnp.sum(t, axis=0), jnp.sum(t * t, axis=0)])

        @pl.when(i == 0)
        def _():
            acc_ref[...] = part

        @pl.when(i > 0)
        def _():
            acc_ref[...] += part

        @pl.when(i == grid - 1)
        def _():
            st_ref[...] = acc_ref[...]

    return pl.pallas_call(
        body,
        grid=(grid,),
        in_specs=[
            pl.BlockSpec((2, bn, 32), lambda i: (0, i, 0)),
            pl.BlockSpec((2, bn, 32), lambda i: (0, i, 0)),
            pl.BlockSpec((bn, 1), lambda i: (i, 0)),
            pl.BlockSpec((1, 64), lambda i: (0, 0)),
        ],
        out_specs=[
            pl.BlockSpec((bn, 64), lambda i: (i, 0)),
            pl.BlockSpec((2, 64), lambda i: (0, 0)),
        ],
        out_shape=[
            jax.ShapeDtypeStruct((n, 64), jnp.float32),
            jax.ShapeDtypeStruct((2, 64), jnp.float32),
        ],
        scratch_shapes=[pltpu.VMEM((2, 64), jnp.float32)],
    )(agg2, h2, dinv, b)


def _tc_pre23(t, st, g, be, w, dinv, n, bn):
    grid = n // bn
    inv_n = 1.0 / n

    def body(t_ref, st_ref, g_ref, be_ref, w_ref, dv_ref, h2_ref):
        m = st_ref[0:1, :] * inv_n
        v = st_ref[1:2, :] * inv_n - m * m
        rstd = lax.rsqrt(v + EPS_)
        act = jnp.maximum((t_ref[...] - m) * rstd * g_ref[...] + be_ref[...],
                          0.0)
        h = jnp.dot(act, w_ref[...],
                    preferred_element_type=jnp.float32) * dv_ref[...]
        h2_ref[0] = h[:, :32]
        h2_ref[1] = h[:, 32:]

    return pl.pallas_call(
        body,
        grid=(grid,),
        in_specs=[
            pl.BlockSpec((bn, 64), lambda i: (i, 0)),
            pl.BlockSpec((2, 64), lambda i: (0, 0)),
            pl.BlockSpec((1, 64), lambda i: (0, 0)),
            pl.BlockSpec((1, 64), lambda i: (0, 0)),
            pl.BlockSpec((64, 64), lambda i: (0, 0)),
            pl.BlockSpec((bn, 1), lambda i: (i, 0)),
        ],
        out_specs=[pl.BlockSpec((2, bn, 32), lambda i: (0, i, 0))],
        out_shape=[jax.ShapeDtypeStruct((2, n, 32), jnp.float32)],
    )(t, st, g, be, w, dinv)[0]


def _tc_final(t, st, g, be, batch2, f1w, f1b, f2w, f2b, n, bn):
    grid = n // bn
    inv_n = 1.0 / n

    def body(t_ref, st_ref, g_ref, be_ref, bt_ref, f1w_ref, f1b_ref,
             f2w_ref, f2b_ref, o_ref, ps_ref, pc_ref):
        i = pl.program_id(0)

        @pl.when(i == 0)
        def _():
            ps_ref[...] = jnp.zeros_like(ps_ref)
            pc_ref[...] = jnp.zeros_like(pc_ref)

        m = st_ref[0:1, :] * inv_n
        v = st_ref[1:2, :] * inv_n - m * m
        rstd = lax.rsqrt(v + EPS_)
        act = jnp.maximum((t_ref[...] - m) * rstd * g_ref[...] + be_ref[...],
                          0.0)
        gid = lax.broadcasted_iota(jnp.int32, (bn, NG_), 1)
        mask = (bt_ref[...] == gid).astype(jnp.float32)
        ps_ref[...] += lax.dot_general(mask, act, (((0,), (0,)), ((), ())),
                                       preferred_element_type=jnp.float32)
        pc_ref[...] += lax.dot_general(mask, jnp.ones((bn, 1), jnp.float32),
                                       (((0,), (0,)), ((), ())),
                                       preferred_element_type=jnp.float32)

        @pl.when(i == grid - 1)
        def _():
            pooled = ps_ref[...] / jnp.maximum(pc_ref[...], 1.0)
            z = jnp.maximum(jnp.dot(pooled, f1w_ref[...],
                                    preferred_element_type=jnp.float32)
                            + f1b_ref[...], 0.0)
            o_ref[...] = jnp.dot(z, f2w_ref[...],
                                 preferred_element_type=jnp.float32) \
                + f2b_ref[...]

    return pl.pallas_call(
        body,
        grid=(grid,),
        in_specs=[
            pl.BlockSpec((bn, 64), lambda i: (i, 0)),
            pl.BlockSpec((2, 64), lambda i: (0, 0)),
            pl.BlockSpec((1, 64), lambda i: (0, 0)),
            pl.BlockSpec((1, 64), lambda i: (0, 0)),
            pl.BlockSpec((bn, 1), lambda i: (i, 0)),
            pl.BlockSpec((64, 64), lambda i: (0, 0)),
            pl.BlockSpec((1, 64), lambda i: (0, 0)),
            pl.BlockSpec((64, 1), lambda i: (0, 0)),
            pl.BlockSpec((1, 1), lambda i: (0, 0)),
        ],
        out_specs=[pl.BlockSpec((NG_, 1), lambda i: (0, 0))],
        out_shape=[jax.ShapeDtypeStruct((NG_, 1), jnp.float32)],
        scratch_shapes=[
            pltpu.VMEM((NG_, 64), jnp.float32),
            pltpu.VMEM((NG_, 1), jnp.float32),
        ],
    )(t, st, g, be, batch2, f1w, f1b, f2w, f2b)[0]


def kernel(x, edge_index, batch, W1, b1, g1, be1, W2, b2, g2, be2,
           W3, b3, g3, be3, fc1_W, fc1_b, fc2_W, fc2_b):
    n = x.shape[0]
    e = edge_index.shape[1]
    bn = 2000
    assert n % bn == 0 and n % NSUB_ == 0

    rrows = _round_up(-(-e // LN_), NSUB_ * BLKG_)
    if (rrows // (NSUB_ * BLKG_)) % 2 == 0:
        rrows += NSUB_ * BLKG_
    epad = rrows * LN_ - e
    src = edge_index[0]
    dst = edge_index[1]
    if epad:
        src = jnp.concatenate([src, jnp.zeros((epad,), jnp.int32)])
        dst = jnp.concatenate([dst, jnp.full((epad,), n, jnp.int32)])
    srcdst = jnp.stack([src.reshape(rrows, LN_), dst.reshape(rrows, LN_)],
                       axis=1)

    na = _round_up(n + 8, LN_)

    degp = _make_sc_deg(n, rrows, na)(srcdst)
    h2, dinv = _tc_pre1(x, degp, W1, n, bn)

    sc_agg = _make_sc_agg(n, rrows, na)
    b1r, b2r, b3r = (b.reshape(1, 64) for b in (b1, b2, b3))
    g1r, g2r, g3r = (g.reshape(1, 64) for g in (g1, g2, g3))
    be1r, be2r, be3r = (b.reshape(1, 64) for b in (be1, be2, be3))

    agg2 = sc_agg(h2, srcdst)
    t1, st1 = _tc_post(agg2, h2, dinv, b1r, n, bn)

    h2 = _tc_pre23(t1, st1, g1r, be1r, W2, dinv, n, bn)
    agg2 = sc_agg(h2, srcdst)
    t2, st2 = _tc_post(agg2, h2, dinv, b2r, n, bn)

    h2 = _tc_pre23(t2, st2, g2r, be2r, W3, dinv, n, bn)
    agg2 = sc_agg(h2, srcdst)
    t3, st3 = _tc_post(agg2, h2, dinv, b3r, n, bn)

    return _tc_final(t3, st3, g3r, be3r, batch.reshape(n, 1),
                     fc1_W, fc1_b.reshape(1, 64), fc2_W,
                     fc2_b.reshape(1, 1), n, bn)
